# Initial kernel scaffold; baseline (speedup 1.0000x reference)
import jax
import jax.numpy as jnp
from jax.experimental import pallas as pl


def kernel(hidden_states, ln_weight, ln_bias, gate_weight, expert_bias):
    raise NotImplementedError("write your pallas kernel here")


# speedup vs baseline: 1.2545x; 1.2545x over previous
import functools

import jax
import jax.numpy as jnp
from jax.experimental import pallas as pl

EPS = 1e-05
NUM_EXPERTS = 64
TOP_K = 8


def _router_block(x_ref, lnw_ref, lnb_ref, gw_ref, eb_ref,
                  probs_ref, idx_ref, logits_ref):
    x = x_ref[...]
    mean = jnp.mean(x, axis=-1, keepdims=True)
    cx = x - mean
    var = jnp.mean(cx * cx, axis=-1, keepdims=True)
    xn = cx * jax.lax.rsqrt(var + EPS)
    xn = xn * lnw_ref[...] + lnb_ref[...]
    logits = jax.lax.dot_general(
        xn, gw_ref[...],
        dimension_numbers=(((1,), (1,)), ((), ())),
        preferred_element_type=jnp.float32,
    )
    logits = logits + eb_ref[...]
    logits_ref[...] = logits
    lmax = jnp.max(logits, axis=-1, keepdims=True)
    e = jnp.exp(logits - lmax)
    probs = e / jnp.sum(e, axis=-1, keepdims=True)
    bt = probs.shape[0]
    iota = jax.lax.broadcasted_iota(jnp.int32, (bt, NUM_EXPERTS), 1)
    work = probs
    vals = []
    idxs = []
    for _ in range(TOP_K):
        m = jnp.max(work, axis=-1, keepdims=True)
        am = jnp.min(jnp.where(work == m, iota, NUM_EXPERTS),
                     axis=-1, keepdims=True)
        vals.append(m)
        idxs.append(am)
        work = jnp.where(iota == am, -jnp.inf, work)
    top_vals = jnp.concatenate(vals, axis=-1)
    top_idx = jnp.concatenate(idxs, axis=-1)
    s = jnp.clip(jnp.sum(top_vals, axis=-1, keepdims=True), EPS, None)
    probs_ref[...] = top_vals / s
    idx_ref[...] = top_idx


@functools.partial(jax.jit, static_argnames=())
def kernel(hidden_states, ln_weight, ln_bias, gate_weight, expert_bias):
    B, S, H = hidden_states.shape
    T = B * S
    E = gate_weight.shape[0]
    x = hidden_states.reshape(T, H)
    lnw = ln_weight.reshape(1, H)
    lnb = ln_bias.reshape(1, H)
    eb = expert_bias.reshape(1, E)

    BT = 256
    grid = (T // BT,)

    probs, idx, logits = pl.pallas_call(
        _router_block,
        grid=grid,
        in_specs=[
            pl.BlockSpec((BT, H), lambda i: (i, 0)),
            pl.BlockSpec((1, H), lambda i: (0, 0)),
            pl.BlockSpec((1, H), lambda i: (0, 0)),
            pl.BlockSpec((E, H), lambda i: (0, 0)),
            pl.BlockSpec((1, E), lambda i: (0, 0)),
        ],
        out_specs=[
            pl.BlockSpec((BT, TOP_K), lambda i: (i, 0)),
            pl.BlockSpec((BT, TOP_K), lambda i: (i, 0)),
            pl.BlockSpec((BT, E), lambda i: (i, 0)),
        ],
        out_shape=[
            jax.ShapeDtypeStruct((T, TOP_K), jnp.float32),
            jax.ShapeDtypeStruct((T, TOP_K), jnp.int32),
            jax.ShapeDtypeStruct((T, E), jnp.float32),
        ],
    )(x, lnw, lnb, gate_weight, eb)
    return probs, idx, logits


# speedup vs baseline: 1.8953x; 1.5108x over previous
import functools

import jax
import jax.numpy as jnp
from jax.experimental import pallas as pl

EPS = 1e-05
NUM_EXPERTS = 64
TOP_K = 8


def _router_block(x_ref, gw_ref, eb_ref, probs_ref, idx_ref, logits_ref):
    x = x_ref[...]
    mean = jnp.mean(x, axis=-1, keepdims=True)
    cx = x - mean
    var = jnp.mean(cx * cx, axis=-1, keepdims=True)
    xn = cx * jax.lax.rsqrt(var + EPS)
    logits = jax.lax.dot_general(
        xn, gw_ref[...],
        dimension_numbers=(((1,), (1,)), ((), ())),
        preferred_element_type=jnp.float32,
    )
    logits = logits + eb_ref[...]
    logits_ref[...] = logits
    lt = logits.T
    lmax = jnp.max(lt, axis=0, keepdims=True)
    e = jnp.exp(lt - lmax)
    bt = lt.shape[1]
    iota = jax.lax.broadcasted_iota(jnp.int32, (NUM_EXPERTS, bt), 0)
    work = e
    vals = []
    idxs = []
    for _ in range(TOP_K):
        m = jnp.max(work, axis=0, keepdims=True)
        am = jnp.min(jnp.where(work == m, iota, NUM_EXPERTS),
                     axis=0, keepdims=True)
        vals.append(m)
        idxs.append(am)
        work = jnp.where(iota == am, -jnp.inf, work)
    top_e = jnp.concatenate(vals, axis=0)
    top_idx = jnp.concatenate(idxs, axis=0)
    s = jnp.sum(top_e, axis=0, keepdims=True)
    probs_ref[...] = (top_e / s).T
    idx_ref[...] = top_idx.T


@functools.partial(jax.jit, static_argnames=())
def kernel(hidden_states, ln_weight, ln_bias, gate_weight, expert_bias):
    B, S, H = hidden_states.shape
    T = B * S
    E = gate_weight.shape[0]
    x = hidden_states.reshape(T, H)
    gw = gate_weight * ln_weight[None, :]
    eb = (expert_bias + gate_weight @ ln_bias).reshape(1, E)

    BT = 256
    grid = (T // BT,)

    probs, idx, logits = pl.pallas_call(
        _router_block,
        grid=grid,
        in_specs=[
            pl.BlockSpec((BT, H), lambda i: (i, 0)),
            pl.BlockSpec((E, H), lambda i: (0, 0)),
            pl.BlockSpec((1, E), lambda i: (0, 0)),
        ],
        out_specs=[
            pl.BlockSpec((BT, TOP_K), lambda i: (i, 0)),
            pl.BlockSpec((BT, TOP_K), lambda i: (i, 0)),
            pl.BlockSpec((BT, E), lambda i: (i, 0)),
        ],
        out_shape=[
            jax.ShapeDtypeStruct((T, TOP_K), jnp.float32),
            jax.ShapeDtypeStruct((T, TOP_K), jnp.int32),
            jax.ShapeDtypeStruct((T, E), jnp.float32),
        ],
    )(x, gw, eb)
    return probs, idx, logits


# speedup vs baseline: 2.2548x; 1.1897x over previous
import functools

import jax
import jax.numpy as jnp
from jax.experimental import pallas as pl

EPS = 1e-05
NUM_EXPERTS = 64
TOP_K = 8


def _router_block(x_ref, gw_ref, eb_ref, probs_ref, idx_ref, logits_ref):
    x = x_ref[...]
    h = x.shape[-1]
    mean = jnp.sum(x, axis=-1, keepdims=True) * (1.0 / h)
    msq = jnp.sum(x * x, axis=-1, keepdims=True) * (1.0 / h)
    var = msq - mean * mean
    xn = (x - mean) * jax.lax.rsqrt(var + EPS)
    logits = jax.lax.dot_general(
        xn, gw_ref[...],
        dimension_numbers=(((1,), (1,)), ((), ())),
        preferred_element_type=jnp.float32,
    )
    logits = logits + eb_ref[...]
    logits_ref[...] = logits
    lt = logits.T
    lmax = jnp.max(lt, axis=0, keepdims=True)
    e = jnp.exp(lt - lmax)
    bt = lt.shape[1]
    iota = jax.lax.broadcasted_iota(jnp.int32, (NUM_EXPERTS, bt), 0)
    work = e
    vals = []
    idxs = []
    for _ in range(TOP_K):
        m = jnp.max(work, axis=0, keepdims=True)
        am = jnp.min(jnp.where(work == m, iota, NUM_EXPERTS),
                     axis=0, keepdims=True)
        vals.append(m)
        idxs.append(am)
        work = jnp.where(iota == am, -jnp.inf, work)
    top_e = jnp.concatenate(vals, axis=0)
    top_idx = jnp.concatenate(idxs, axis=0)
    s = jnp.sum(top_e, axis=0, keepdims=True)
    probs_ref[...] = (top_e / s).T
    idx_ref[...] = top_idx.T


@functools.partial(jax.jit, static_argnames=())
def kernel(hidden_states, ln_weight, ln_bias, gate_weight, expert_bias):
    B, S, H = hidden_states.shape
    T = B * S
    E = gate_weight.shape[0]
    x = hidden_states.reshape(T, H)
    gw = gate_weight * ln_weight[None, :]
    eb = (expert_bias + gate_weight @ ln_bias).reshape(1, E)

    BT = 512
    grid = (T // BT,)

    probs, idx, logits = pl.pallas_call(
        _router_block,
        grid=grid,
        in_specs=[
            pl.BlockSpec((BT, H), lambda i: (i, 0)),
            pl.BlockSpec((E, H), lambda i: (0, 0)),
            pl.BlockSpec((1, E), lambda i: (0, 0)),
        ],
        out_specs=[
            pl.BlockSpec((BT, TOP_K), lambda i: (i, 0)),
            pl.BlockSpec((BT, TOP_K), lambda i: (i, 0)),
            pl.BlockSpec((BT, E), lambda i: (i, 0)),
        ],
        out_shape=[
            jax.ShapeDtypeStruct((T, TOP_K), jnp.float32),
            jax.ShapeDtypeStruct((T, TOP_K), jnp.int32),
            jax.ShapeDtypeStruct((T, E), jnp.float32),
        ],
    )(x, gw, eb)
    return probs, idx, logits


# speedup vs baseline: 2.2600x; 1.0023x over previous
import functools

import jax
import jax.numpy as jnp
from jax.experimental import pallas as pl

EPS = 1e-05
NUM_EXPERTS = 64
TOP_K = 8


def _router_block(x_ref, gw_ref, eb_ref, probs_ref, idx_ref, logits_ref):
    x = x_ref[...]
    h = x.shape[-1]
    mean = jnp.sum(x, axis=-1, keepdims=True) * (1.0 / h)
    msq = jnp.sum(x * x, axis=-1, keepdims=True) * (1.0 / h)
    var = msq - mean * mean
    xn = (x - mean) * jax.lax.rsqrt(var + EPS)
    logits = jax.lax.dot_general(
        xn.astype(jnp.bfloat16), gw_ref[...].astype(jnp.bfloat16),
        dimension_numbers=(((1,), (1,)), ((), ())),
        preferred_element_type=jnp.float32,
    )
    logits = logits + eb_ref[...]
    logits_ref[...] = logits
    lt = logits.T
    lmax = jnp.max(lt, axis=0, keepdims=True)
    e = jnp.exp(lt - lmax)
    bt = lt.shape[1]
    iota = jax.lax.broadcasted_iota(jnp.int32, (NUM_EXPERTS, bt), 0)
    work = e
    vals = []
    idxs = []
    for _ in range(TOP_K):
        m = jnp.max(work, axis=0, keepdims=True)
        am = jnp.min(jnp.where(work == m, iota, NUM_EXPERTS),
                     axis=0, keepdims=True)
        vals.append(m)
        idxs.append(am)
        work = jnp.where(iota == am, -jnp.inf, work)
    top_e = jnp.concatenate(vals, axis=0)
    top_idx = jnp.concatenate(idxs, axis=0)
    s = jnp.sum(top_e, axis=0, keepdims=True)
    probs_ref[...] = (top_e / s).T
    idx_ref[...] = top_idx.T


@functools.partial(jax.jit, static_argnames=())
def kernel(hidden_states, ln_weight, ln_bias, gate_weight, expert_bias):
    B, S, H = hidden_states.shape
    T = B * S
    E = gate_weight.shape[0]
    x = hidden_states.reshape(T, H)
    gw = gate_weight * ln_weight[None, :]
    eb = (expert_bias + gate_weight @ ln_bias).reshape(1, E)

    BT = 512
    grid = (T // BT,)

    probs, idx, logits = pl.pallas_call(
        _router_block,
        grid=grid,
        in_specs=[
            pl.BlockSpec((BT, H), lambda i: (i, 0)),
            pl.BlockSpec((E, H), lambda i: (0, 0)),
            pl.BlockSpec((1, E), lambda i: (0, 0)),
        ],
        out_specs=[
            pl.BlockSpec((BT, TOP_K), lambda i: (i, 0)),
            pl.BlockSpec((BT, TOP_K), lambda i: (i, 0)),
            pl.BlockSpec((BT, E), lambda i: (i, 0)),
        ],
        out_shape=[
            jax.ShapeDtypeStruct((T, TOP_K), jnp.float32),
            jax.ShapeDtypeStruct((T, TOP_K), jnp.int32),
            jax.ShapeDtypeStruct((T, E), jnp.float32),
        ],
    )(x, gw, eb)
    return probs, idx, logits


# speedup vs baseline: 2.4548x; 1.0862x over previous
import functools

import jax
import jax.numpy as jnp
from jax.experimental import pallas as pl

EPS = 1e-05
NUM_EXPERTS = 64
TOP_K = 8


def _router_block(x_ref, gw_ref, eb_ref, probs_ref, idx_ref, logits_ref):
    x = x_ref[...]
    h = x.shape[-1]
    mean = jnp.sum(x, axis=-1, keepdims=True) * (1.0 / h)
    msq = jnp.sum(x * x, axis=-1, keepdims=True) * (1.0 / h)
    var = msq - mean * mean
    xn = (x - mean) * jax.lax.rsqrt(var + EPS)
    logits = jax.lax.dot_general(
        xn.astype(jnp.bfloat16), gw_ref[...].astype(jnp.bfloat16),
        dimension_numbers=(((1,), (1,)), ((), ())),
        preferred_element_type=jnp.float32,
    )
    logits = logits + eb_ref[...]
    logits_ref[...] = logits
    lt = logits.T
    lmax = jnp.max(lt, axis=0, keepdims=True)
    e = jnp.exp(lt - lmax)
    bt = lt.shape[1]
    iota = jax.lax.broadcasted_iota(jnp.int32, (NUM_EXPERTS, bt), 0)
    work = e
    vals = []
    idxs = []
    for _ in range(TOP_K):
        m = jnp.max(work, axis=0, keepdims=True)
        am = jnp.min(jnp.where(work == m, iota, NUM_EXPERTS),
                     axis=0, keepdims=True)
        vals.append(m)
        idxs.append(am)
        work = jnp.where(iota == am, -jnp.inf, work)
    top_e = jnp.concatenate(vals, axis=0)
    top_idx = jnp.concatenate(idxs, axis=0)
    s = jnp.sum(top_e, axis=0, keepdims=True)
    probs_ref[...] = (top_e / s).T
    idx_ref[...] = top_idx.T


@functools.partial(jax.jit, static_argnames=())
def kernel(hidden_states, ln_weight, ln_bias, gate_weight, expert_bias):
    B, S, H = hidden_states.shape
    T = B * S
    E = gate_weight.shape[0]
    x = hidden_states.reshape(T, H)
    gw = gate_weight * ln_weight[None, :]
    eb = (expert_bias + gate_weight @ ln_bias).reshape(1, E)

    BT = 1024
    grid = (T // BT,)

    probs, idx, logits = pl.pallas_call(
        _router_block,
        grid=grid,
        in_specs=[
            pl.BlockSpec((BT, H), lambda i: (i, 0)),
            pl.BlockSpec((E, H), lambda i: (0, 0)),
            pl.BlockSpec((1, E), lambda i: (0, 0)),
        ],
        out_specs=[
            pl.BlockSpec((BT, TOP_K), lambda i: (i, 0)),
            pl.BlockSpec((BT, TOP_K), lambda i: (i, 0)),
            pl.BlockSpec((BT, E), lambda i: (i, 0)),
        ],
        out_shape=[
            jax.ShapeDtypeStruct((T, TOP_K), jnp.float32),
            jax.ShapeDtypeStruct((T, TOP_K), jnp.int32),
            jax.ShapeDtypeStruct((T, E), jnp.float32),
        ],
    )(x, gw, eb)
    return probs, idx, logits
